# Initial kernel scaffold; baseline (speedup 1.0000x reference)
import jax
import jax.numpy as jnp
from jax.experimental import pallas as pl


def kernel(x, embedding_table):
    raise NotImplementedError("write your pallas kernel here")


# speedup vs baseline: 1.8717x; 1.8717x over previous
import functools

import jax
import jax.numpy as jnp
from jax import lax
from jax.experimental import pallas as pl
from jax.experimental.pallas import tpu as pltpu
from jax.experimental.pallas import tpu_sc as plsc

VOCAB = 1000000
D = 64
B = 16384 * 50
NC, NS = 2, 16
NW = NC * NS
PER_W = B // NW
CHUNK = 128
K = 4
NCH = PER_W // CHUNK
G = NCH // K


def _gather_body(idx_hbm, table_hbm, out_hbm, idx_v, rows_v, sem_g, sem_w0,
                 sem_w1):
    wid = lax.axis_index("s") * NC + lax.axis_index("c")
    out_base = wid * PER_W

    pltpu.sync_copy(idx_hbm.at[wid], idx_v)

    sem_w = (sem_w0, sem_w1)

    @pl.loop(0, G, step=2)
    def _(j):
        for p in range(2):
            g = j + p

            @pl.when(g >= 2)
            def _():
                for b in range(K):
                    pltpu.make_async_copy(
                        rows_v.at[p, b], out_hbm.at[pl.ds(0, CHUNK)],
                        sem_w[p]).wait()

            gathers = []
            for b in range(K):
                c = g * K + b
                gathers.append(
                    pltpu.async_copy(table_hbm.at[idx_v.at[c]],
                                     rows_v.at[p, b], sem_g))
            for h in gathers:
                h.wait()

            for b in range(K):
                c = g * K + b
                pltpu.async_copy(rows_v.at[p, b],
                                 out_hbm.at[pl.ds(out_base + c * CHUNK,
                                                  CHUNK)], sem_w[p])

    for p in range(2):
        for b in range(K):
            pltpu.make_async_copy(rows_v.at[p, b],
                                  out_hbm.at[pl.ds(0, CHUNK)],
                                  sem_w[p]).wait()


@jax.jit
def _lookup(idx, table):
    mesh = plsc.VectorSubcoreMesh(core_axis_name="c", subcore_axis_name="s")
    f = functools.partial(
        pl.kernel,
        mesh=mesh,
        out_type=jax.ShapeDtypeStruct((B, D), jnp.float32),
        scratch_types=[
            pltpu.VMEM((NCH, CHUNK), jnp.int32),
            pltpu.VMEM((2, K, CHUNK, D), jnp.float32),
            pltpu.SemaphoreType.DMA,
            pltpu.SemaphoreType.DMA,
            pltpu.SemaphoreType.DMA,
        ],
        compiler_params=pltpu.CompilerParams(use_tc_tiling_on_sc=False),
    )(_gather_body)
    return f(idx, table)


def kernel(x, embedding_table):
    idx = x.astype(jnp.int32).reshape(NW, NCH, CHUNK)
    out = _lookup(idx, embedding_table)
    return out.reshape(x.shape[0], x.shape[1], D)
